# baseline (speedup 1.0000x reference)
import dataclasses
import functools

import numpy as np

import jax
import jax.numpy as jnp
from jax import lax
from jax.experimental import pallas as pl
from jax.experimental.pallas import tpu as pltpu
from jax.experimental.pallas import tpu_sc as plsc

N = 10000
E = 160000
C = 256
HALF = 128
E_DIM = 16

NT = 16
EPT = E // NT
K = 80
CPT = EPT // K
ROWS_PT = 624
ZROWS = 208
REM_BASE = NT * ROWS_PT
REM = N - REM_BASE


_BE = 8000


def _eproj_body(ea_ref, we2_ref, be2_ref, o_ref):
    e = jnp.dot(ea_ref[...], we2_ref[...], preferred_element_type=jnp.float32)
    e = (e + be2_ref[...]).astype(jnp.bfloat16).astype(jnp.float32)
    ei = jax.lax.bitcast_convert_type(e, jnp.int32)
    for h in range(2):
        pieces = []
        for side in range(2):
            seg = ei[:, side * C + h * HALF: side * C + (h + 1) * HALF]
            pieces.append(jnp.bitwise_or(
                jax.lax.shift_right_logical(seg[:, : HALF // 2], 16),
                jnp.bitwise_and(seg[:, HALF // 2:], jnp.int32(-65536))))
        o_ref[h] = jnp.concatenate(pieces, axis=1)


_eproj = pl.pallas_call(
    _eproj_body,
    grid=(E // _BE,),
    in_specs=[
        pl.BlockSpec((_BE // 2, 2 * E_DIM), lambda i: (i, 0)),
        pl.BlockSpec((2 * E_DIM, 2 * C), lambda i: (0, 0)),
        pl.BlockSpec((1, 2 * C), lambda i: (0, 0)),
    ],
    out_specs=pl.BlockSpec((2, _BE // 2, HALF), lambda i: (0, i, 0)),
    out_shape=jax.ShapeDtypeStruct((2, E // 2, HALF), jnp.int32),
)

_BX = 2000


def _xsplit_body(x_ref, o_ref):
    o_ref[0] = x_ref[:, :HALF]
    o_ref[1] = x_ref[:, HALF:]


_xsplit = pl.pallas_call(
    _xsplit_body,
    grid=(N // _BX,),
    in_specs=[pl.BlockSpec((_BX, C), lambda i: (i, 0))],
    out_specs=pl.BlockSpec((2, _BX, HALF), lambda i: (0, i, 0)),
    out_shape=jax.ShapeDtypeStruct((2, N, HALF), jnp.float32),
)


_sc_mesh = plsc.VectorSubcoreMesh(core_axis_name="c", subcore_axis_name="s")

_sc_params = pltpu.CompilerParams()
if "needs_layout_passes" in pltpu.CompilerParams.__dataclass_fields__:
    _sc_params = dataclasses.replace(_sc_params, needs_layout_passes=False)


NBUF = 3
NRING = 8
ECH = K // 2


@functools.partial(
    pl.kernel,
    out_type=jax.ShapeDtypeStruct((2, N, HALF), jnp.float32),
    mesh=_sc_mesh,
    compiler_params=_sc_params,
    scratch_types=[
        pltpu.VMEM_SHARED((N, HALF), jnp.float32),
        pltpu.VMEM((NRING, K), jnp.int32),
        pltpu.VMEM((NRING, K), jnp.int32),
        pltpu.VMEM((NBUF, K, HALF), jnp.float32),
        pltpu.VMEM((NBUF, ECH, HALF), jnp.int32),
        pltpu.SemaphoreType.DMA((NBUF,)),
        pltpu.SemaphoreType.DMA((NBUF,)),
        pltpu.SemaphoreType.DMA((NBUF,)),
        pltpu.SemaphoreType.DMA((NRING,)),
    ],
)
def _sc_msg(x_hbm, e_hbm, src_hbm, dst_hbm, out_hbm, acc, sring, dring, mbuf,
            ebuf, gsem, esem, ssem, isem):
    c = lax.axis_index("c")
    s = lax.axis_index("s")
    zero = jnp.zeros((16,), jnp.float32)
    _HIMASK = jnp.int32(-65536)
    base_e = s * EPT

    @pl.loop(0, K)
    def _zrow(r):
        for j in range(0, HALF, 16):
            mbuf[0, r, pl.ds(j, 16)] = zero

    @pl.loop(0, ROWS_PT // K)
    def _zdma(j):
        pltpu.sync_copy(mbuf.at[0], acc.at[pl.ds(s * ROWS_PT + j * K, K)])

    _zrem = ROWS_PT - (ROWS_PT // K) * K
    if _zrem:
        pltpu.sync_copy(mbuf.at[0].at[pl.ds(0, _zrem)],
                        acc.at[pl.ds(s * ROWS_PT + ROWS_PT - _zrem, _zrem)])

    @pl.when(s == NT - 1)
    def _ztail():
        pltpu.sync_copy(mbuf.at[0].at[pl.ds(0, REM)], acc.at[pl.ds(REM_BASE, REM)])

    plsc.subcore_barrier()

    def issue_idx(i):
        r = lax.rem(i, NRING)
        row0 = base_e + i * K
        pltpu.async_copy(src_hbm.at[pl.ds(row0, K)], sring.at[r], isem.at[r])
        pltpu.async_copy(dst_hbm.at[pl.ds(row0, K)], dring.at[r], isem.at[r])

    def wait_idx(i):
        r = lax.rem(i, NRING)
        row0 = base_e + i * K
        pltpu.make_async_copy(src_hbm.at[pl.ds(row0, K)], sring.at[r],
                              isem.at[r]).wait()
        pltpu.make_async_copy(dst_hbm.at[pl.ds(row0, K)], dring.at[r],
                              isem.at[r]).wait()

    def issue_ge(i):
        b = lax.rem(i, NBUF)
        r = lax.rem(i, NRING)
        erow0 = pl.multiple_of((base_e + i * K) // 2, 8)
        pltpu.async_copy(x_hbm.at[c].at[sring.at[r]], mbuf.at[b], gsem.at[b])
        pltpu.async_copy(e_hbm.at[c].at[pl.ds(erow0, ECH)], ebuf.at[b],
                         esem.at[b])

    def wait_ge(i):
        b = lax.rem(i, NBUF)
        r = lax.rem(i, NRING)
        erow0 = pl.multiple_of((base_e + i * K) // 2, 8)
        pltpu.make_async_copy(x_hbm.at[c].at[sring.at[r]], mbuf.at[b],
                              gsem.at[b]).wait()
        pltpu.make_async_copy(e_hbm.at[c].at[pl.ds(erow0, ECH)], ebuf.at[b],
                              esem.at[b]).wait()

    def issue_scatter(i):
        b = lax.rem(i, NBUF)
        r = lax.rem(i, NRING)
        pltpu.async_copy(mbuf.at[b], acc.at[dring.at[r]], ssem.at[b], add=True)

    def wait_scatter(i):
        b = lax.rem(i, NBUF)
        r = lax.rem(i, NRING)
        pltpu.make_async_copy(mbuf.at[b], acc.at[dring.at[r]],
                              ssem.at[b]).wait()

    def compute(i):
        b = lax.rem(i, NBUF)

        @plsc.parallel_loop(0, ECH, unroll=8)
        def _erow(er):
            for side in range(2):
                r = 2 * er + side
                for j in range(0, HALF // 2, 16):
                    w = ebuf[b, er, pl.ds(side * (HALF // 2) + j, 16)]
                    ea = plsc.bitcast(jnp.left_shift(w, 16), jnp.float32)
                    eb = plsc.bitcast(jnp.bitwise_and(w, _HIMASK), jnp.float32)
                    v0 = mbuf[b, r, pl.ds(j, 16)] + ea
                    v1 = mbuf[b, r, pl.ds(HALF // 2 + j, 16)] + eb
                    mbuf[b, r, pl.ds(j, 16)] = jnp.maximum(v0, zero)
                    mbuf[b, r, pl.ds(HALF // 2 + j, 16)] = jnp.maximum(v1, zero)

    for i in range(3):
        issue_idx(i)

    @pl.loop(0, CPT + 2)
    def _step(t):
        @pl.when(t >= 2)
        def _s2():
            i = t - 2
            wait_ge(i)
            compute(i)
            issue_scatter(i)

        @pl.when(t < CPT)
        def _s0():
            wait_idx(t)

            @pl.when(t >= NBUF)
            def _():
                wait_scatter(t - NBUF)

            issue_ge(t)

        @pl.when(t + 3 < CPT)
        def _si():
            issue_idx(t + 3)

    for i in range(CPT - NBUF, CPT):
        wait_scatter(i)

    plsc.subcore_barrier()

    @pl.loop(0, ROWS_PT // ZROWS)
    def _drain(j):
        r0 = s * ROWS_PT + j * ZROWS
        pltpu.sync_copy(acc.at[pl.ds(r0, ZROWS)], out_hbm.at[c].at[pl.ds(r0, ZROWS)])

    @pl.when(s == NT - 1)
    def _dtail():
        pltpu.sync_copy(acc.at[pl.ds(REM_BASE, REM)],
                        out_hbm.at[c].at[pl.ds(REM_BASE, REM)])


_BN = 1000


def _mlp_body(x_ref, agg_ref, w1_ref, b1_ref, w2_ref, b2_ref, o_ref):
    h = jnp.concatenate([agg_ref[0], agg_ref[1]], axis=-1) + x_ref[...]
    t = jnp.dot(h, w1_ref[...], preferred_element_type=jnp.float32)
    t = jnp.maximum(t + b1_ref[...], 0.0)
    o = jnp.dot(t, w2_ref[...], preferred_element_type=jnp.float32)
    o_ref[...] = o + b2_ref[...]


_mlp = pl.pallas_call(
    _mlp_body,
    grid=(N // _BN,),
    in_specs=[
        pl.BlockSpec((_BN, C), lambda i: (i, 0)),
        pl.BlockSpec((2, _BN, HALF), lambda i: (0, i, 0)),
        pl.BlockSpec((C, 2 * C), lambda i: (0, 0)),
        pl.BlockSpec((1, 2 * C), lambda i: (0, 0)),
        pl.BlockSpec((2 * C, C), lambda i: (0, 0)),
        pl.BlockSpec((1, C), lambda i: (0, 0)),
    ],
    out_specs=pl.BlockSpec((_BN, C), lambda i: (i, 0)),
    out_shape=jax.ShapeDtypeStruct((N, C), jnp.float32),
)


def kernel(node_feats, edge_index, edge_attr, We, be, W1, b1, W2, b2):
    src = edge_index[0]
    dst = edge_index[1]
    x_stacked = _xsplit(node_feats)
    We2 = (jnp.zeros((2 * E_DIM, 2 * C), We.dtype)
           .at[:E_DIM, :C].set(We).at[E_DIM:, C:].set(We))
    be2 = jnp.concatenate([be, be]).reshape(1, 2 * C)
    e_stacked = _eproj(edge_attr.reshape(E // 2, 2 * E_DIM), We2, be2)
    agg_stacked = _sc_msg(x_stacked, e_stacked, src, dst)
    return _mlp(node_feats, agg_stacked, W1, b1.reshape(1, 2 * C),
                W2, b2.reshape(1, C))


# speedup vs baseline: 1.0313x; 1.0313x over previous
import dataclasses
import functools

import numpy as np

import jax
import jax.numpy as jnp
from jax import lax
from jax.experimental import pallas as pl
from jax.experimental.pallas import tpu as pltpu
from jax.experimental.pallas import tpu_sc as plsc

N = 10000
E = 160000
C = 256
HALF = 128
E_DIM = 16

NT = 16
EPT = E // NT
K = 80
CPT = EPT // K
ROWS_PT = 624
ZROWS = 208
REM_BASE = NT * ROWS_PT
REM = N - REM_BASE


_BE = 8000


def _eproj_body(ea_ref, we2_ref, be2_ref, o_ref):
    e = jnp.dot(ea_ref[...], we2_ref[...], preferred_element_type=jnp.float32)
    e = (e + be2_ref[...]).astype(jnp.bfloat16).astype(jnp.float32)
    ei = jax.lax.bitcast_convert_type(e, jnp.int32)
    for h in range(2):
        pieces = []
        for side in range(2):
            seg = ei[:, side * C + h * HALF: side * C + (h + 1) * HALF]
            pieces.append(jnp.bitwise_or(
                jax.lax.shift_right_logical(seg[:, : HALF // 2], 16),
                jnp.bitwise_and(seg[:, HALF // 2:], jnp.int32(-65536))))
        o_ref[h] = jnp.concatenate(pieces, axis=1)


_eproj = pl.pallas_call(
    _eproj_body,
    grid=(E // _BE,),
    in_specs=[
        pl.BlockSpec((_BE // 2, 2 * E_DIM), lambda i: (i, 0)),
        pl.BlockSpec((2 * E_DIM, 2 * C), lambda i: (0, 0)),
        pl.BlockSpec((1, 2 * C), lambda i: (0, 0)),
    ],
    out_specs=pl.BlockSpec((2, _BE // 2, HALF), lambda i: (0, i, 0)),
    out_shape=jax.ShapeDtypeStruct((2, E // 2, HALF), jnp.int32),
)

_BX = 2000


def _xsplit_body(x_ref, o_ref):
    o_ref[0] = x_ref[:, :HALF]
    o_ref[1] = x_ref[:, HALF:]


_xsplit = pl.pallas_call(
    _xsplit_body,
    grid=(N // _BX,),
    in_specs=[pl.BlockSpec((_BX, C), lambda i: (i, 0))],
    out_specs=pl.BlockSpec((2, _BX, HALF), lambda i: (0, i, 0)),
    out_shape=jax.ShapeDtypeStruct((2, N, HALF), jnp.float32),
)


_sc_mesh = plsc.VectorSubcoreMesh(core_axis_name="c", subcore_axis_name="s")

_sc_params = pltpu.CompilerParams()
if "needs_layout_passes" in pltpu.CompilerParams.__dataclass_fields__:
    _sc_params = dataclasses.replace(_sc_params, needs_layout_passes=False)


NBUF = 3
NRING = 8
ECH = K // 2


@functools.partial(
    pl.kernel,
    out_type=jax.ShapeDtypeStruct((2, N, HALF), jnp.float32),
    mesh=_sc_mesh,
    compiler_params=_sc_params,
    scratch_types=[
        pltpu.VMEM_SHARED((N, HALF), jnp.float32),
        pltpu.VMEM((NRING, K), jnp.int32),
        pltpu.VMEM((NRING, K), jnp.int32),
        pltpu.VMEM((NBUF, K, HALF), jnp.float32),
        pltpu.VMEM((NBUF, ECH, HALF), jnp.int32),
        pltpu.SemaphoreType.DMA((NBUF,)),
        pltpu.SemaphoreType.DMA((NBUF,)),
        pltpu.SemaphoreType.DMA((NBUF,)),
        pltpu.SemaphoreType.DMA((NRING,)),
    ],
)
def _sc_msg(x_hbm, e_hbm, src_hbm, dst_hbm, out_hbm, acc, sring, dring, mbuf,
            ebuf, gsem, esem, ssem, isem):
    c = lax.axis_index("c")
    s = lax.axis_index("s")
    zero = jnp.zeros((16,), jnp.float32)
    _HIMASK = jnp.int32(-65536)
    base_e = s * EPT

    @pl.loop(0, K)
    def _zrow(r):
        for j in range(0, HALF, 16):
            mbuf[0, r, pl.ds(j, 16)] = zero

    @pl.loop(0, ROWS_PT // K)
    def _zdma(j):
        pltpu.sync_copy(mbuf.at[0], acc.at[pl.ds(s * ROWS_PT + j * K, K)])

    _zrem = ROWS_PT - (ROWS_PT // K) * K
    if _zrem:
        pltpu.sync_copy(mbuf.at[0].at[pl.ds(0, _zrem)],
                        acc.at[pl.ds(s * ROWS_PT + ROWS_PT - _zrem, _zrem)])

    @pl.when(s == NT - 1)
    def _ztail():
        pltpu.sync_copy(mbuf.at[0].at[pl.ds(0, REM)], acc.at[pl.ds(REM_BASE, REM)])

    plsc.subcore_barrier()

    def issue_idx(i):
        r = lax.rem(i, NRING)
        row0 = base_e + i * K
        pltpu.async_copy(src_hbm.at[pl.ds(row0, K)], sring.at[r], isem.at[r])
        pltpu.async_copy(dst_hbm.at[pl.ds(row0, K)], dring.at[r], isem.at[r])

    def wait_idx(i):
        r = lax.rem(i, NRING)
        row0 = base_e + i * K
        pltpu.make_async_copy(src_hbm.at[pl.ds(row0, K)], sring.at[r],
                              isem.at[r]).wait()
        pltpu.make_async_copy(dst_hbm.at[pl.ds(row0, K)], dring.at[r],
                              isem.at[r]).wait()

    def issue_ge(i):
        b = lax.rem(i, NBUF)
        r = lax.rem(i, NRING)
        erow0 = pl.multiple_of((base_e + i * K) // 2, 8)
        pltpu.async_copy(x_hbm.at[c].at[sring.at[r]], mbuf.at[b], gsem.at[b])
        pltpu.async_copy(e_hbm.at[c].at[pl.ds(erow0, ECH)], ebuf.at[b],
                         esem.at[b])

    def wait_ge(i):
        b = lax.rem(i, NBUF)
        r = lax.rem(i, NRING)
        erow0 = pl.multiple_of((base_e + i * K) // 2, 8)
        pltpu.make_async_copy(x_hbm.at[c].at[sring.at[r]], mbuf.at[b],
                              gsem.at[b]).wait()
        pltpu.make_async_copy(e_hbm.at[c].at[pl.ds(erow0, ECH)], ebuf.at[b],
                              esem.at[b]).wait()

    def issue_scatter(i):
        b = lax.rem(i, NBUF)
        r = lax.rem(i, NRING)
        pltpu.async_copy(mbuf.at[b], acc.at[dring.at[r]], ssem.at[b], add=True)

    def wait_scatter(i):
        b = lax.rem(i, NBUF)
        r = lax.rem(i, NRING)
        pltpu.make_async_copy(mbuf.at[b], acc.at[dring.at[r]],
                              ssem.at[b]).wait()

    def compute(i):
        b = lax.rem(i, NBUF)

        @plsc.parallel_loop(0, ECH, unroll=4)
        def _erow(er):
            for side in range(2):
                r = 2 * er + side
                for j in range(0, HALF // 2, 16):
                    w = ebuf[b, er, pl.ds(side * (HALF // 2) + j, 16)]
                    ea = plsc.bitcast(jnp.left_shift(w, 16), jnp.float32)
                    eb = plsc.bitcast(jnp.bitwise_and(w, _HIMASK), jnp.float32)
                    v0 = mbuf[b, r, pl.ds(j, 16)] + ea
                    v1 = mbuf[b, r, pl.ds(HALF // 2 + j, 16)] + eb
                    mbuf[b, r, pl.ds(j, 16)] = jnp.maximum(v0, zero)
                    mbuf[b, r, pl.ds(HALF // 2 + j, 16)] = jnp.maximum(v1, zero)

    for i in range(3):
        issue_idx(i)

    @pl.loop(0, CPT + 2)
    def _step(t):
        @pl.when(t >= 2)
        def _s2():
            i = t - 2
            wait_ge(i)
            compute(i)
            issue_scatter(i)

        @pl.when(t < CPT)
        def _s0():
            wait_idx(t)

            @pl.when(t >= NBUF)
            def _():
                wait_scatter(t - NBUF)

            issue_ge(t)

        @pl.when(t + 3 < CPT)
        def _si():
            issue_idx(t + 3)

    for i in range(CPT - NBUF, CPT):
        wait_scatter(i)

    plsc.subcore_barrier()

    @pl.loop(0, ROWS_PT // ZROWS)
    def _drain(j):
        r0 = s * ROWS_PT + j * ZROWS
        pltpu.sync_copy(acc.at[pl.ds(r0, ZROWS)], out_hbm.at[c].at[pl.ds(r0, ZROWS)])

    @pl.when(s == NT - 1)
    def _dtail():
        pltpu.sync_copy(acc.at[pl.ds(REM_BASE, REM)],
                        out_hbm.at[c].at[pl.ds(REM_BASE, REM)])


_BN = 1000


def _mlp_body(x_ref, agg_ref, w1_ref, b1_ref, w2_ref, b2_ref, o_ref):
    h = jnp.concatenate([agg_ref[0], agg_ref[1]], axis=-1) + x_ref[...]
    t = jnp.dot(h, w1_ref[...], preferred_element_type=jnp.float32)
    t = jnp.maximum(t + b1_ref[...], 0.0)
    o = jnp.dot(t, w2_ref[...], preferred_element_type=jnp.float32)
    o_ref[...] = o + b2_ref[...]


_mlp = pl.pallas_call(
    _mlp_body,
    grid=(N // _BN,),
    in_specs=[
        pl.BlockSpec((_BN, C), lambda i: (i, 0)),
        pl.BlockSpec((2, _BN, HALF), lambda i: (0, i, 0)),
        pl.BlockSpec((C, 2 * C), lambda i: (0, 0)),
        pl.BlockSpec((1, 2 * C), lambda i: (0, 0)),
        pl.BlockSpec((2 * C, C), lambda i: (0, 0)),
        pl.BlockSpec((1, C), lambda i: (0, 0)),
    ],
    out_specs=pl.BlockSpec((_BN, C), lambda i: (i, 0)),
    out_shape=jax.ShapeDtypeStruct((N, C), jnp.float32),
)


def kernel(node_feats, edge_index, edge_attr, We, be, W1, b1, W2, b2):
    src = edge_index[0]
    dst = edge_index[1]
    x_stacked = _xsplit(node_feats)
    We2 = (jnp.zeros((2 * E_DIM, 2 * C), We.dtype)
           .at[:E_DIM, :C].set(We).at[E_DIM:, C:].set(We))
    be2 = jnp.concatenate([be, be]).reshape(1, 2 * C)
    e_stacked = _eproj(edge_attr.reshape(E // 2, 2 * E_DIM), We2, be2)
    agg_stacked = _sc_msg(x_stacked, e_stacked, src, dst)
    return _mlp(node_feats, agg_stacked, W1, b1.reshape(1, 2 * C),
                W2, b2.reshape(1, C))


# speedup vs baseline: 1.0322x; 1.0009x over previous
import dataclasses
import functools

import jax
import jax.numpy as jnp
from jax import lax
from jax.experimental import pallas as pl
from jax.experimental.pallas import tpu as pltpu
from jax.experimental.pallas import tpu_sc as plsc

N = 10000
E = 160000
C = 256
HALF = 128
E_DIM = 16

NT = 16
EPT = E // NT
K = 80
CPT = EPT // K
ROWS_PT = 624
ZROWS = 208
REM_BASE = NT * ROWS_PT
REM = N - REM_BASE


_BE = 8000


def _eproj_body(ea_ref, we2_ref, be2_ref, o_ref):
    e = jnp.dot(ea_ref[...], we2_ref[...], preferred_element_type=jnp.float32)
    e = (e + be2_ref[...]).astype(jnp.bfloat16).astype(jnp.float32)
    ei = jax.lax.bitcast_convert_type(e, jnp.int32)
    for h in range(2):
        pieces = []
        for side in range(2):
            seg = ei[:, side * C + h * HALF: side * C + (h + 1) * HALF]
            pieces.append(jnp.bitwise_or(
                jax.lax.shift_right_logical(seg[:, : HALF // 2], 16),
                jnp.bitwise_and(seg[:, HALF // 2:], jnp.int32(-65536))))
        o_ref[h] = jnp.concatenate(pieces, axis=1)


_eproj = pl.pallas_call(
    _eproj_body,
    grid=(E // _BE,),
    in_specs=[
        pl.BlockSpec((_BE // 2, 2 * E_DIM), lambda i: (i, 0)),
        pl.BlockSpec((2 * E_DIM, 2 * C), lambda i: (0, 0)),
        pl.BlockSpec((1, 2 * C), lambda i: (0, 0)),
    ],
    out_specs=pl.BlockSpec((2, _BE // 2, HALF), lambda i: (0, i, 0)),
    out_shape=jax.ShapeDtypeStruct((2, E // 2, HALF), jnp.int32),
)

_BX = 2000


def _xsplit_body(x_ref, o_ref):
    o_ref[0] = x_ref[:, :HALF]
    o_ref[1] = x_ref[:, HALF:]


_xsplit = pl.pallas_call(
    _xsplit_body,
    grid=(N // _BX,),
    in_specs=[pl.BlockSpec((_BX, C), lambda i: (i, 0))],
    out_specs=pl.BlockSpec((2, _BX, HALF), lambda i: (0, i, 0)),
    out_shape=jax.ShapeDtypeStruct((2, N, HALF), jnp.float32),
)


_sc_mesh = plsc.VectorSubcoreMesh(core_axis_name="c", subcore_axis_name="s")

_sc_params = pltpu.CompilerParams()
if "needs_layout_passes" in pltpu.CompilerParams.__dataclass_fields__:
    _sc_params = dataclasses.replace(_sc_params, needs_layout_passes=False)


NBUF = 3
NRING = 8
ECH = K // 2


@functools.partial(
    pl.kernel,
    out_type=jax.ShapeDtypeStruct((2, N, HALF), jnp.float32),
    mesh=_sc_mesh,
    compiler_params=_sc_params,
    scratch_types=[
        pltpu.VMEM_SHARED((N, HALF), jnp.float32),
        pltpu.VMEM((NRING, K), jnp.int32),
        pltpu.VMEM((NRING, K), jnp.int32),
        pltpu.VMEM((NBUF, K, HALF), jnp.float32),
        pltpu.VMEM((NBUF, ECH, HALF), jnp.int32),
        pltpu.SemaphoreType.DMA((NBUF,)),
        pltpu.SemaphoreType.DMA((NBUF,)),
        pltpu.SemaphoreType.DMA((NBUF,)),
        pltpu.SemaphoreType.DMA((NRING,)),
    ],
)
def _sc_msg(x_hbm, e_hbm, src_hbm, dst_hbm, out_hbm, acc, sring, dring, mbuf,
            ebuf, gsem, esem, ssem, isem):
    c = lax.axis_index("c")
    s = lax.axis_index("s")
    zero = jnp.zeros((16,), jnp.float32)
    _HIMASK = jnp.int32(-65536)
    base_e = s * EPT

    @pl.loop(0, K)
    def _zrow(r):
        for j in range(0, HALF, 16):
            mbuf[0, r, pl.ds(j, 16)] = zero

    @pl.loop(0, ROWS_PT // K)
    def _zdma(j):
        pltpu.sync_copy(mbuf.at[0], acc.at[pl.ds(s * ROWS_PT + j * K, K)])

    _zrem = ROWS_PT - (ROWS_PT // K) * K
    if _zrem:
        pltpu.sync_copy(mbuf.at[0].at[pl.ds(0, _zrem)],
                        acc.at[pl.ds(s * ROWS_PT + ROWS_PT - _zrem, _zrem)])

    @pl.when(s == NT - 1)
    def _ztail():
        pltpu.sync_copy(mbuf.at[0].at[pl.ds(0, REM)], acc.at[pl.ds(REM_BASE, REM)])

    plsc.subcore_barrier()

    def issue_idx(i):
        r = lax.rem(i, NRING)
        row0 = base_e + i * K
        pltpu.async_copy(src_hbm.at[pl.ds(row0, K)], sring.at[r], isem.at[r])
        pltpu.async_copy(dst_hbm.at[pl.ds(row0, K)], dring.at[r], isem.at[r])

    def wait_idx(i):
        r = lax.rem(i, NRING)
        row0 = base_e + i * K
        pltpu.make_async_copy(src_hbm.at[pl.ds(row0, K)], sring.at[r],
                              isem.at[r]).wait()
        pltpu.make_async_copy(dst_hbm.at[pl.ds(row0, K)], dring.at[r],
                              isem.at[r]).wait()

    def issue_ge(i):
        b = lax.rem(i, NBUF)
        r = lax.rem(i, NRING)
        erow0 = pl.multiple_of((base_e + i * K) // 2, 8)
        pltpu.async_copy(x_hbm.at[c].at[sring.at[r]], mbuf.at[b], gsem.at[b])
        pltpu.async_copy(e_hbm.at[c].at[pl.ds(erow0, ECH)], ebuf.at[b],
                         esem.at[b])

    def wait_ge(i):
        b = lax.rem(i, NBUF)
        r = lax.rem(i, NRING)
        erow0 = pl.multiple_of((base_e + i * K) // 2, 8)
        pltpu.make_async_copy(x_hbm.at[c].at[sring.at[r]], mbuf.at[b],
                              gsem.at[b]).wait()
        pltpu.make_async_copy(e_hbm.at[c].at[pl.ds(erow0, ECH)], ebuf.at[b],
                              esem.at[b]).wait()

    def issue_scatter(i):
        b = lax.rem(i, NBUF)
        r = lax.rem(i, NRING)
        pltpu.async_copy(mbuf.at[b], acc.at[dring.at[r]], ssem.at[b], add=True)

    def wait_scatter(i):
        b = lax.rem(i, NBUF)
        r = lax.rem(i, NRING)
        pltpu.make_async_copy(mbuf.at[b], acc.at[dring.at[r]],
                              ssem.at[b]).wait()

    def compute(i):
        b = lax.rem(i, NBUF)

        @plsc.parallel_loop(0, ECH, unroll=4)
        def _erow(er):
            for side in range(2):
                r = 2 * er + side
                for j in range(0, HALF // 2, 16):
                    w = ebuf[b, er, pl.ds(side * (HALF // 2) + j, 16)]
                    ea = plsc.bitcast(jnp.left_shift(w, 16), jnp.float32)
                    eb = plsc.bitcast(jnp.bitwise_and(w, _HIMASK), jnp.float32)
                    v0 = mbuf[b, r, pl.ds(j, 16)] + ea
                    v1 = mbuf[b, r, pl.ds(HALF // 2 + j, 16)] + eb
                    mbuf[b, r, pl.ds(j, 16)] = jnp.maximum(v0, zero)
                    mbuf[b, r, pl.ds(HALF // 2 + j, 16)] = jnp.maximum(v1, zero)

    for i in range(3):
        issue_idx(i)

    @pl.loop(0, CPT + 2)
    def _step(t):
        @pl.when(t >= 2)
        def _s2():
            i = t - 2
            wait_ge(i)
            compute(i)
            issue_scatter(i)

        @pl.when(t < CPT)
        def _s0():
            wait_idx(t)

            @pl.when(t >= NBUF)
            def _():
                wait_scatter(t - NBUF)

            issue_ge(t)

        @pl.when(t + 3 < CPT)
        def _si():
            issue_idx(t + 3)

    for i in range(CPT - NBUF, CPT):
        wait_scatter(i)

    plsc.subcore_barrier()

    @pl.loop(0, ROWS_PT // ZROWS)
    def _drain(j):
        r0 = s * ROWS_PT + j * ZROWS
        pltpu.sync_copy(acc.at[pl.ds(r0, ZROWS)], out_hbm.at[c].at[pl.ds(r0, ZROWS)])

    @pl.when(s == NT - 1)
    def _dtail():
        pltpu.sync_copy(acc.at[pl.ds(REM_BASE, REM)],
                        out_hbm.at[c].at[pl.ds(REM_BASE, REM)])


_BN = 1000


def _mlp_body(x_ref, agg_ref, w1_ref, b1_ref, w2_ref, b2_ref, o_ref):
    h = jnp.concatenate([agg_ref[0], agg_ref[1]], axis=-1) + x_ref[...]
    t = jnp.dot(h, w1_ref[...], preferred_element_type=jnp.float32)
    t = jnp.maximum(t + b1_ref[...], 0.0)
    o = jnp.dot(t, w2_ref[...], preferred_element_type=jnp.float32)
    o_ref[...] = o + b2_ref[...]


_mlp = pl.pallas_call(
    _mlp_body,
    grid=(N // _BN,),
    in_specs=[
        pl.BlockSpec((_BN, C), lambda i: (i, 0)),
        pl.BlockSpec((2, _BN, HALF), lambda i: (0, i, 0)),
        pl.BlockSpec((C, 2 * C), lambda i: (0, 0)),
        pl.BlockSpec((1, 2 * C), lambda i: (0, 0)),
        pl.BlockSpec((2 * C, C), lambda i: (0, 0)),
        pl.BlockSpec((1, C), lambda i: (0, 0)),
    ],
    out_specs=pl.BlockSpec((_BN, C), lambda i: (i, 0)),
    out_shape=jax.ShapeDtypeStruct((N, C), jnp.float32),
)


def kernel(node_feats, edge_index, edge_attr, We, be, W1, b1, W2, b2):
    src = edge_index[0]
    dst = edge_index[1]
    x_stacked = _xsplit(node_feats)
    We2 = (jnp.zeros((2 * E_DIM, 2 * C), We.dtype)
           .at[:E_DIM, :C].set(We).at[E_DIM:, C:].set(We))
    be2 = jnp.concatenate([be, be]).reshape(1, 2 * C)
    e_stacked = _eproj(edge_attr.reshape(E // 2, 2 * E_DIM), We2, be2)
    agg_stacked = _sc_msg(x_stacked, e_stacked, src, dst)
    return _mlp(node_feats, agg_stacked, W1, b1.reshape(1, 2 * C),
                W2, b2.reshape(1, C))
